# baseline (speedup 1.0000x reference)
import jax
import jax.numpy as jnp
from jax import lax
from jax.experimental import pallas as pl
from jax.experimental.pallas import tpu as pltpu
from jax.experimental.pallas import tpu_sc as plsc

BATCH = 4
SEQ_LEN = 4096
D_MODEL = 2560
NUM_TOKENS = BATCH * SEQ_LEN

NUM_CORES = 2
NUM_SUBCORES = 16
NUM_WORKERS = NUM_CORES * NUM_SUBCORES

N_TC = 4096
N_SC = NUM_TOKENS - N_TC
IDS_PER_WORKER = N_SC // NUM_WORKERS

C = 8
NBUF = 4
NUM_CHUNKS = IDS_PER_WORKER // C

G = 8


def _sc_gather(table, ids_sc):
    mesh = plsc.VectorSubcoreMesh(core_axis_name="c", subcore_axis_name="s")

    @pl.kernel(
        out_type=jax.ShapeDtypeStruct((N_SC, D_MODEL), jnp.float32),
        mesh=mesh,
        scratch_types=(
            [pltpu.VMEM((IDS_PER_WORKER,), jnp.int32)]
            + [pltpu.VMEM((C, D_MODEL), jnp.float32) for _ in range(NBUF)]
            + [pltpu.SemaphoreType.DMA for _ in range(2 * NBUF)]
        ),
    )
    def gather_kernel(table_hbm, ids_hbm, out_hbm, idx_v, *bufs_sems):
        bufs = bufs_sems[:NBUF]
        gsems = bufs_sems[NBUF:2 * NBUF]
        osems = bufs_sems[2 * NBUF:]

        wid = lax.axis_index("s") * NUM_CORES + lax.axis_index("c")
        base = wid * IDS_PER_WORKER
        pltpu.sync_copy(ids_hbm.at[pl.ds(base, IDS_PER_WORKER)], idx_v)

        def gather_start(chunk, b):
            pltpu.async_copy(
                table_hbm.at[idx_v.at[pl.ds(chunk * C, C)]], bufs[b], gsems[b]
            )

        for b in range(NBUF - 1):
            gather_start(b, b)

        @pl.loop(0, NUM_CHUNKS, step=NBUF)
        def _(i):
            for b in range(NBUF):
                chunk = i + b
                bp = (b - 1) % NBUF
                nxt = chunk + NBUF - 1

                @pl.when(jnp.logical_and(chunk >= 1, nxt < NUM_CHUNKS))
                def _():
                    pltpu.make_async_copy(
                        bufs[bp], out_hbm.at[pl.ds(base, C)], osems[bp]
                    ).wait()

                @pl.when(nxt < NUM_CHUNKS)
                def _():
                    gather_start(nxt, bp)

                pltpu.make_async_copy(
                    table_hbm.at[idx_v.at[pl.ds(chunk * C, C)]],
                    bufs[b], gsems[b],
                ).wait()
                pltpu.async_copy(
                    bufs[b], out_hbm.at[pl.ds(base + chunk * C, C)], osems[b]
                )

        for b in range(NBUF):
            pltpu.make_async_copy(
                bufs[b], out_hbm.at[pl.ds(base, C)], osems[b]
            ).wait()

    return gather_kernel(table, ids_sc)


def _tc_gather(table, ids_tc):
    table3 = table.reshape(VOCAB := table.shape[0], 1, D_MODEL)

    def body(ids_ref, *refs):
        del ids_ref
        ins = refs[:G]
        out = refs[G]
        for g in range(G):
            out[g, :] = ins[g][0, 0, :]

    def make_in_map(g):
        return lambda i, ids: (ids[i * G + g], 0, 0)

    grid_spec = pltpu.PrefetchScalarGridSpec(
        num_scalar_prefetch=1,
        grid=(N_TC // G,),
        in_specs=[
            pl.BlockSpec((1, 1, D_MODEL), make_in_map(g)) for g in range(G)
        ],
        out_specs=pl.BlockSpec((G, D_MODEL), lambda i, ids: (i, 0)),
    )
    return pl.pallas_call(
        body,
        grid_spec=grid_spec,
        out_shape=jax.ShapeDtypeStruct((N_TC, D_MODEL), jnp.float32),
    )(ids_tc, *([table3] * G))


def kernel(token_ids, table):
    ids_flat = token_ids.reshape(NUM_TOKENS).astype(jnp.int32)
    sc_out = _sc_gather(table, ids_flat[:N_SC])
    tc_out = _tc_gather(table, ids_flat[N_SC:])
    out = jnp.concatenate([sc_out, tc_out], axis=0)
    return out.reshape(BATCH, SEQ_LEN, D_MODEL)


# speedup vs baseline: 12.3455x; 12.3455x over previous
import jax
import jax.numpy as jnp
from jax import lax
from jax.experimental import pallas as pl
from jax.experimental.pallas import tpu as pltpu
from jax.experimental.pallas import tpu_sc as plsc

BATCH = 4
SEQ_LEN = 4096
D_MODEL = 2560
NUM_TOKENS = BATCH * SEQ_LEN

NUM_CORES = 2
NUM_SUBCORES = 16
NUM_WORKERS = NUM_CORES * NUM_SUBCORES
IDS_PER_WORKER = NUM_TOKENS // NUM_WORKERS
WORKERS_PER_BATCH = SEQ_LEN // IDS_PER_WORKER

C = 8
NBUF = 4
NUM_CHUNKS = IDS_PER_WORKER // C


def kernel(token_ids, table):
    mesh = plsc.VectorSubcoreMesh(core_axis_name="c", subcore_axis_name="s")

    @pl.kernel(
        out_type=jax.ShapeDtypeStruct((BATCH, SEQ_LEN, D_MODEL), jnp.float32),
        mesh=mesh,
        scratch_types=(
            [pltpu.VMEM((IDS_PER_WORKER,), jnp.int32)]
            + [pltpu.VMEM((C, D_MODEL), jnp.float32) for _ in range(NBUF)]
            + [pltpu.SemaphoreType.DMA for _ in range(2 * NBUF)]
        ),
    )
    def gather_kernel(table_hbm, ids_hbm, out_hbm, idx_v, *bufs_sems):
        bufs = bufs_sems[:NBUF]
        gsems = bufs_sems[NBUF:2 * NBUF]
        osems = bufs_sems[2 * NBUF:]

        wid = lax.axis_index("s") * NUM_CORES + lax.axis_index("c")
        batch = wid // WORKERS_PER_BATCH
        scol = (wid % WORKERS_PER_BATCH) * IDS_PER_WORKER
        pltpu.sync_copy(ids_hbm.at[batch, pl.ds(scol, IDS_PER_WORKER)], idx_v)

        def gather_start(chunk, b):
            pltpu.async_copy(
                table_hbm.at[idx_v.at[pl.ds(chunk * C, C)]], bufs[b], gsems[b]
            )

        for b in range(NBUF - 1):
            gather_start(b, b)

        @pl.loop(0, NUM_CHUNKS, step=NBUF)
        def _(i):
            for b in range(NBUF):
                chunk = i + b
                bp = (b - 1) % NBUF
                nxt = chunk + NBUF - 1

                @pl.when(jnp.logical_and(chunk >= 1, nxt < NUM_CHUNKS))
                def _():
                    pltpu.make_async_copy(
                        bufs[bp], out_hbm.at[batch, pl.ds(scol, C)], osems[bp]
                    ).wait()

                @pl.when(nxt < NUM_CHUNKS)
                def _():
                    gather_start(nxt, bp)

                pltpu.make_async_copy(
                    table_hbm.at[idx_v.at[pl.ds(chunk * C, C)]],
                    bufs[b], gsems[b],
                ).wait()
                pltpu.async_copy(
                    bufs[b],
                    out_hbm.at[batch, pl.ds(scol + chunk * C, C)],
                    osems[b],
                )

        for b in range(NBUF):
            pltpu.make_async_copy(
                bufs[b], out_hbm.at[batch, pl.ds(scol, C)], osems[b]
            ).wait()

    return gather_kernel(table, token_ids)


# speedup vs baseline: 12.3490x; 1.0003x over previous
import jax
import jax.numpy as jnp
from jax import lax
from jax.experimental import pallas as pl
from jax.experimental.pallas import tpu as pltpu
from jax.experimental.pallas import tpu_sc as plsc

BATCH = 4
SEQ_LEN = 4096
D_MODEL = 2560
NUM_TOKENS = BATCH * SEQ_LEN

NUM_CORES = 2
NUM_SUBCORES = 16
NUM_WORKERS = NUM_CORES * NUM_SUBCORES
IDS_PER_WORKER = NUM_TOKENS // NUM_WORKERS
WORKERS_PER_BATCH = SEQ_LEN // IDS_PER_WORKER

C = 8
NBUF = 4
NUM_CHUNKS = IDS_PER_WORKER // C


def kernel(token_ids, table):
    token_ids = token_ids.astype(jnp.int32)
    mesh = plsc.VectorSubcoreMesh(core_axis_name="c", subcore_axis_name="s")

    @pl.kernel(
        out_type=jax.ShapeDtypeStruct((BATCH, SEQ_LEN, D_MODEL), jnp.float32),
        mesh=mesh,
        scratch_types=(
            [pltpu.VMEM((IDS_PER_WORKER,), jnp.int32)]
            + [pltpu.VMEM((C, D_MODEL), jnp.float32) for _ in range(NBUF)]
            + [pltpu.SemaphoreType.DMA for _ in range(2 * NBUF)]
        ),
    )
    def gather_kernel(table_hbm, ids_hbm, out_hbm, idx_v, *bufs_sems):
        bufs = bufs_sems[:NBUF]
        gsems = bufs_sems[NBUF:2 * NBUF]
        osems = bufs_sems[2 * NBUF:]

        wid = lax.axis_index("s") * NUM_CORES + lax.axis_index("c")
        batch = wid // WORKERS_PER_BATCH
        scol = (wid % WORKERS_PER_BATCH) * IDS_PER_WORKER
        pltpu.sync_copy(ids_hbm.at[batch, pl.ds(scol, IDS_PER_WORKER)], idx_v)

        def gather_start(chunk, b):
            pltpu.async_copy(
                table_hbm.at[idx_v.at[pl.ds(chunk * C, C)]], bufs[b], gsems[b]
            )

        for b in range(NBUF - 1):
            gather_start(b, b)

        @pl.loop(0, NUM_CHUNKS, step=NBUF)
        def _(i):
            for b in range(NBUF):
                chunk = i + b
                bp = (b - 1) % NBUF
                nxt = chunk + NBUF - 1

                @pl.when(jnp.logical_and(chunk >= 1, nxt < NUM_CHUNKS))
                def _():
                    pltpu.make_async_copy(
                        bufs[bp], out_hbm.at[batch, pl.ds(scol, C)], osems[bp]
                    ).wait()

                @pl.when(nxt < NUM_CHUNKS)
                def _():
                    gather_start(nxt, bp)

                pltpu.make_async_copy(
                    table_hbm.at[idx_v.at[pl.ds(chunk * C, C)]],
                    bufs[b], gsems[b],
                ).wait()
                pltpu.async_copy(
                    bufs[b],
                    out_hbm.at[batch, pl.ds(scol + chunk * C, C)],
                    osems[b],
                )

        for b in range(NBUF):
            pltpu.make_async_copy(
                bufs[b], out_hbm.at[batch, pl.ds(scol, C)], osems[b]
            ).wait()

    return gather_kernel(table, token_ids)
